# baseline (speedup 1.0000x reference)
import functools

import jax
import jax.numpy as jnp
import numpy as np
from jax import lax
from jax.experimental import pallas as pl
from jax.experimental.pallas import tpu as pltpu
from jax.experimental.pallas import tpu_sc as plsc

_D = 64
_SCALE = 8.0
_NW = 32
_CHUNK = 128
_SEQ = 200


def _positional_encoding(length, depth):
    half = depth / 2
    positions = np.arange(length)[:, None]
    depths = np.arange(half)[None, :] / half
    angle_rates = 1 / 10000**depths
    angle_rads = positions * angle_rates
    return np.concatenate(
        [np.sin(angle_rads), np.cos(angle_rads)], axis=-1
    ).astype(np.float32)


def _sc_gather_fused(table_r, idx_flat, pos2):
    n = idx_flat.shape[0]
    per_w = n // _NW
    n_chunks = per_w // _CHUNK
    mesh = plsc.VectorSubcoreMesh(core_axis_name="c", subcore_axis_name="s")

    @functools.partial(
        pl.kernel,
        out_type=jax.ShapeDtypeStruct((n, _D), jnp.float32),
        mesh=mesh,
        scratch_types=[
            pltpu.VMEM((_CHUNK,), jnp.int32),
            pltpu.VMEM((_CHUNK,), jnp.int32),
            pltpu.VMEM((_CHUNK, _D), jnp.float32),
            pltpu.VMEM((_CHUNK, _D), jnp.float32),
            pltpu.VMEM((2 * _SEQ, _D), jnp.float32),
            pltpu.SemaphoreType.DMA,
            pltpu.SemaphoreType.DMA,
            pltpu.SemaphoreType.DMA,
            pltpu.SemaphoreType.DMA,
            pltpu.SemaphoreType.DMA,
            pltpu.SemaphoreType.DMA,
            pltpu.SemaphoreType.DMA,
        ],
    )
    def k(
        table_hbm, idx_hbm, pos_hbm, out_hbm,
        idx0, idx1, rows0, rows1, pos_v,
        psem, isem0, isem1, gsem0, gsem1, osem0, osem1,
    ):
        wid = lax.axis_index("s") * 2 + lax.axis_index("c")
        base = wid * per_w
        idx_b = (idx0, idx1)
        rows_b = (rows0, rows1)
        isem_b = (isem0, isem1)
        gsem_b = (gsem0, gsem1)
        osem_b = (osem0, osem1)

        pltpu.async_copy(pos_hbm, pos_v, psem).wait()
        pltpu.async_copy(idx_hbm.at[pl.ds(base, _CHUNK)], idx0, isem0)
        pltpu.async_copy(
            idx_hbm.at[pl.ds(base + _CHUNK, _CHUNK)], idx1, isem1
        )

        def wait_idx(b):
            pltpu.make_async_copy(
                idx_hbm.at[pl.ds(0, _CHUNK)], idx_b[b], isem_b[b]
            ).wait()

        def drain_rows(b):
            pltpu.make_async_copy(
                out_hbm.at[pl.ds(0, _CHUNK)], rows_b[b], gsem_b[b]
            ).wait()

        def wait_out(b):
            pltpu.make_async_copy(
                rows_b[b], out_hbm.at[pl.ds(0, _CHUNK)], osem_b[b]
            ).wait()

        def prefetch_idx(b, c):
            @pl.when(c + 2 < n_chunks)
            def _():
                pltpu.async_copy(
                    idx_hbm.at[pl.ds(base + (c + 2) * _CHUNK, _CHUNK)],
                    idx_b[b], isem_b[b],
                )

        def enqueue_group(b, g):
            vec = idx_b[b][pl.ds(g * 16, 16)]
            for kk in range(16):
                i = vec[kk]
                pltpu.async_copy(
                    table_hbm.at[i >> 3, pl.ds(i & 7, 1), :],
                    rows_b[b].at[pl.ds(g * 16 + kk, 1), :],
                    gsem_b[b],
                )

        def fma_group(b, seq0, g):
            for kk in range(16):
                j = g * 16 + kk
                r = rows_b[b].at[j]
                p = pos_v.at[seq0 + j]
                for t in range(_D // 16):
                    sl = pl.ds(t * 16, 16)
                    r[sl] = r[sl] * _SCALE + p[sl]

        def writeback(b, c):
            pltpu.async_copy(
                rows_b[b],
                out_hbm.at[pl.ds(base + c * _CHUNK, _CHUNK)],
                osem_b[b],
            )

        wait_idx(0)

        @pl.loop(0, _CHUNK // 16)
        def _(g):
            enqueue_group(0, g)

        prefetch_idx(0, 0)

        @pl.loop(0, (n_chunks - 1) // 2)
        def _(co):
            for m in range(2):
                c = 1 + co * 2 + m
                b = (1 + m) % 2

                @pl.when(c >= 2)
                def _():
                    wait_out(b)

                wait_idx(b)
                drain_rows(1 - b)
                seq0 = lax.rem((c - 1) * _CHUNK, _SEQ)

                @pl.loop(0, _CHUNK // 16)
                def _(g):
                    enqueue_group(b, g)
                    fma_group(1 - b, seq0, g)

                writeback(1 - b, c - 1)
                prefetch_idx(b, c)

        last = n_chunks - 1
        wait_out(1)
        wait_idx(1)
        drain_rows(0)
        seq_prev = (last - 1) * _CHUNK % _SEQ

        @pl.loop(0, _CHUNK // 16)
        def _(g):
            enqueue_group(1, g)
            fma_group(0, seq_prev, g)

        writeback(0, last - 1)

        drain_rows(1)
        seq_last = last * _CHUNK % _SEQ

        @pl.loop(0, _CHUNK // 16)
        def _(g):
            fma_group(1, seq_last, g)

        writeback(1, last)
        wait_out(0)
        wait_out(1)

    return k(table_r, idx_flat, pos2)


def kernel(x, table):
    batch, seq = x.shape
    pos = jnp.asarray(_positional_encoding(seq, _D))
    pos2 = jnp.concatenate([pos, pos], axis=0)
    idx_flat = x.reshape(batch * seq)
    table_r = table.reshape(table.shape[0] // 8, 8, _D)
    g = _sc_gather_fused(table_r, idx_flat, pos2)
    return g.reshape(batch, seq, _D)


# speedup vs baseline: 1.2287x; 1.2287x over previous
import functools

import jax
import jax.numpy as jnp
import numpy as np
from jax import lax
from jax.experimental import pallas as pl
from jax.experimental.pallas import tpu as pltpu
from jax.experimental.pallas import tpu_sc as plsc

_D = 64
_SCALE = 8.0
_NW = 32
_CHUNK = 200
_SEQ = 200


def _positional_encoding(length, depth):
    half = depth / 2
    positions = np.arange(length)[:, None]
    depths = np.arange(half)[None, :] / half
    angle_rates = 1 / 10000**depths
    angle_rads = positions * angle_rates
    return np.concatenate(
        [np.sin(angle_rads), np.cos(angle_rads)], axis=-1
    ).astype(np.float32)


def _sc_gather_fused(table_r, idx_flat, pos):
    n = idx_flat.shape[0]
    per_w = n // _NW
    n_chunks = per_w // _CHUNK
    mesh = plsc.VectorSubcoreMesh(core_axis_name="c", subcore_axis_name="s")

    @functools.partial(
        pl.kernel,
        out_type=jax.ShapeDtypeStruct((n, _D), jnp.float32),
        mesh=mesh,
        scratch_types=[
            pltpu.VMEM((_CHUNK,), jnp.int32),
            pltpu.VMEM((_CHUNK,), jnp.int32),
            pltpu.VMEM((_CHUNK, _D), jnp.float32),
            pltpu.VMEM((_CHUNK, _D), jnp.float32),
            pltpu.VMEM((_SEQ, _D), jnp.float32),
            pltpu.SemaphoreType.DMA,
            pltpu.SemaphoreType.DMA,
            pltpu.SemaphoreType.DMA,
            pltpu.SemaphoreType.DMA,
            pltpu.SemaphoreType.DMA,
            pltpu.SemaphoreType.DMA,
            pltpu.SemaphoreType.DMA,
        ],
    )
    def k(
        table_hbm, idx_hbm, pos_hbm, out_hbm,
        idx0, idx1, rows0, rows1, pos_v,
        psem, isem0, isem1, gsem0, gsem1, osem0, osem1,
    ):
        wid = lax.axis_index("s") * 2 + lax.axis_index("c")
        base = wid * per_w
        idx_b = (idx0, idx1)
        rows_b = (rows0, rows1)
        isem_b = (isem0, isem1)
        gsem_b = (gsem0, gsem1)
        osem_b = (osem0, osem1)

        pltpu.async_copy(pos_hbm, pos_v, psem).wait()
        pltpu.async_copy(idx_hbm.at[pl.ds(base, _CHUNK)], idx0, isem0)
        pltpu.async_copy(
            idx_hbm.at[pl.ds(base + _CHUNK, _CHUNK)], idx1, isem1
        )

        def wait_idx(b):
            pltpu.make_async_copy(
                idx_hbm.at[pl.ds(0, _CHUNK)], idx_b[b], isem_b[b]
            ).wait()

        def drain_rows(b):
            pltpu.make_async_copy(
                out_hbm.at[pl.ds(0, _CHUNK)], rows_b[b], gsem_b[b]
            ).wait()

        def wait_out(b):
            pltpu.make_async_copy(
                rows_b[b], out_hbm.at[pl.ds(0, _CHUNK)], osem_b[b]
            ).wait()

        def enqueue_gathers(b):
            @pl.loop(0, _CHUNK // 16)
            def _(g):
                vec = idx_b[b][pl.ds(g * 16, 16)]
                for kk in range(16):
                    i = vec[kk]
                    pltpu.async_copy(
                        table_hbm.at[i >> 3, pl.ds(i & 7, 1), :],
                        rows_b[b].at[pl.ds(g * 16 + kk, 1), :],
                        gsem_b[b],
                    )

            vec = idx_b[b][pl.ds(_CHUNK - 16, 16)]
            for kk in range(8, 16):
                i = vec[kk]
                pltpu.async_copy(
                    table_hbm.at[i >> 3, pl.ds(i & 7, 1), :],
                    rows_b[b].at[pl.ds(_CHUNK - 16 + kk, 1), :],
                    gsem_b[b],
                )

        def fma_and_writeback(b, c):
            @pl.loop(0, _CHUNK)
            def _(j):
                r = rows_b[b].at[j]
                p = pos_v.at[j]
                for t in range(_D // 16):
                    sl = pl.ds(t * 16, 16)
                    r[sl] = r[sl] * _SCALE + p[sl]

            pltpu.async_copy(
                rows_b[b],
                out_hbm.at[pl.ds(base + c * _CHUNK, _CHUNK)],
                osem_b[b],
            )

        @pl.loop(0, n_chunks // 2)
        def _(co):
            for b in range(2):
                c = co * 2 + b

                @pl.when(c >= 2)
                def _():
                    wait_out(b)

                wait_idx(b)
                enqueue_gathers(b)

                @pl.when(c + 2 < n_chunks)
                def _():
                    pltpu.async_copy(
                        idx_hbm.at[pl.ds(base + (c + 2) * _CHUNK, _CHUNK)],
                        idx_b[b], isem_b[b],
                    )

                @pl.when(c >= 1)
                def _():
                    drain_rows(1 - b)
                    fma_and_writeback(1 - b, c - 1)

        drain_rows(1)
        fma_and_writeback(1, n_chunks - 1)
        wait_out(0)
        wait_out(1)

    return k(table_r, idx_flat, pos)


def kernel(x, table):
    batch, seq = x.shape
    pos = jnp.asarray(_positional_encoding(seq, _D))
    idx_flat = x.reshape(batch * seq)
    table_r = table.reshape(table.shape[0] // 8, 8, _D)
    g = _sc_gather_fused(table_r, idx_flat, pos)
    return g.reshape(batch, seq, _D)
